# baseline (speedup 1.0000x reference)
import functools
import math

import jax
import jax.numpy as jnp
from jax import lax
from jax.experimental import pallas as pl
from jax.experimental.pallas import tpu as pltpu
from jax.experimental.pallas import tpu_sc as plsc

TPERC = 0.45
LANES = 16
NC, NS = 2, 16
NW = NC * NS
RING = 2
PER_W_BY_CORE = (184, 448)


def _selection_network(deg, out_lo, out_hi):
    net = []

    def merge(lo, n, r):
        m = r * 2
        if m < n:
            merge(lo, n, m)
            merge(lo + r, n, m)
            for i in range(lo + r, lo + n - r, m):
                net.append((i, i + r))
        else:
            net.append((lo, lo + r))

    def sort(lo, n):
        if n > 1:
            m = n // 2
            sort(lo, m)
            sort(lo + m, m)
            merge(lo, n, 1)

    sort(0, deg)

    state = {i: None for i in range(deg)}
    for w in range(out_lo, out_hi):
        state[w] = 'sum'
    kept = []
    for (i, j) in reversed(net):
        si, sj = state[i], state[j]
        if si is None and sj is None:
            continue
        if si == 'sum' and sj == 'sum':
            continue
        kept.append((i, j, si is not None, sj is not None))
        state[i] = 'exact'
        state[j] = 'exact'
    kept.reverse()
    return kept


def _matmul_body(x_ref, w_ref, h_ref):
    h_ref[...] = lax.dot_general(
        x_ref[...], w_ref[...],
        dimension_numbers=(((1,), (1,)), ((), ())),
        preferred_element_type=jnp.float32,
    )


def _linear(x, W):
    n, d_in = x.shape
    d_out = W.shape[0]
    blk = 1000
    assert n % blk == 0
    return pl.pallas_call(
        _matmul_body,
        grid=(n // blk,),
        in_specs=[
            pl.BlockSpec((blk, d_in), lambda i: (i, 0)),
            pl.BlockSpec((d_out, d_in), lambda i: (0, 0)),
        ],
        out_specs=pl.BlockSpec((blk, d_out), lambda i: (i, 0)),
        out_shape=jax.ShapeDtypeStruct((n, d_out), jnp.float32),
    )(x, W)


def _make_sc_trimmed_mean(n_pad, d, deg, remove):
    keep = deg - 2 * remove
    net = _selection_network(deg, remove, deg - remove)
    nb = max(1, 128 // deg)
    p0, p1 = PER_W_BY_CORE
    assert NS * (p0 + p1) == n_pad
    per_w_max = max(p0, p1)
    for p_c in (p0, p1):
        assert p_c % (nb * RING) == 0
    chunks = d // LANES
    idx_per_batch = nb * deg

    mesh = plsc.VectorSubcoreMesh(core_axis_name="c", subcore_axis_name="s")

    scratch = [pltpu.VMEM((per_w_max * deg,), jnp.int32)]
    scratch += [pltpu.VMEM((idx_per_batch,), jnp.int32) for _ in range(RING)]
    scratch += [pltpu.VMEM((idx_per_batch, d), jnp.float32)
                for _ in range(RING)]
    scratch += [pltpu.VMEM((per_w_max, d), jnp.float32)]
    scratch += [pltpu.SemaphoreType.DMA for _ in range(RING)]

    @functools.partial(
        pl.kernel,
        mesh=mesh,
        out_type=jax.ShapeDtypeStruct((n_pad, d), jnp.float32),
        scratch_types=scratch,
    )
    def sc_kernel(h_hbm, nbrs_hbm, out_hbm, idx_all, *rest):
        ibs = rest[0:RING]
        rows_bufs = rest[RING:2 * RING]
        out_all = rest[2 * RING]
        sems = rest[2 * RING + 1:]

        cid = lax.axis_index("c")
        sid = lax.axis_index("s")

        def fill_idx(b, ib):
            for k in range(idx_per_batch // LANES):
                ib[pl.ds(k * LANES, LANES)] = (
                    idx_all[pl.ds(b * idx_per_batch + k * LANES, LANES)])

        def start_gather(ib, rows, sem):
            pltpu.make_async_copy(h_hbm.at[ib], rows, sem).start()

        def run_pipeline(node_base, per_w):
            nbatch = per_w // nb

            pltpu.sync_copy(
                nbrs_hbm.at[pl.ds(node_base * deg, per_w * deg)],
                idx_all.at[pl.ds(0, per_w * deg)])

            def compute_batch(b, rows):
                def chunk_body(t, carry):
                    i = t // chunks
                    c = (t % chunks) * LANES
                    v = [rows[i * deg + k, pl.ds(c, LANES)]
                         for k in range(deg)]
                    for (a, bb, need_min, need_max) in net:
                        lo, hi = v[a], v[bb]
                        if need_min:
                            v[a] = jnp.minimum(lo, hi)
                        if need_max:
                            v[bb] = jnp.maximum(lo, hi)
                    acc = v[remove]
                    for w in range(remove + 1, deg - remove):
                        acc = acc + v[w]
                    out_all[b * nb + i, pl.ds(c, LANES)] = acc * (1.0 / keep)
                    return carry
                lax.fori_loop(0, nb * chunks, chunk_body, 0)

            for pb in range(RING - 1):
                fill_idx(pb, ibs[pb])
                start_gather(ibs[pb], rows_bufs[pb], sems[pb])

            def outer(bi, carry):
                for p in range(RING):
                    b = bi * RING + p
                    pn = (p + RING - 1) % RING

                    @pl.when(b + RING - 1 < nbatch)
                    def _():
                        fill_idx(b + RING - 1, ibs[pn])
                        start_gather(ibs[pn], rows_bufs[pn], sems[pn])

                    pltpu.make_async_copy(
                        h_hbm.at[ibs[p]], rows_bufs[p], sems[p]).wait()
                    compute_batch(b, rows_bufs[p])
                return carry

            lax.fori_loop(0, nbatch // RING, outer, 0)

            pltpu.sync_copy(
                out_all.at[pl.ds(0, per_w)],
                out_hbm.at[pl.ds(node_base, per_w)])

        @pl.when(cid == 0)
        def _():
            run_pipeline(sid * p0, p0)

        @pl.when(cid == 1)
        def _():
            run_pipeline(NS * p0 + sid * p1, p1)

    return sc_kernel


@jax.jit
def kernel(x, nbrs, W):
    n, d_in = x.shape
    d_out = W.shape[0]
    deg = nbrs.shape[1]
    remove = math.floor(deg * TPERC)

    h = _linear(x, W)

    n_pad = NS * sum(PER_W_BY_CORE)
    assert n_pad >= n
    nbrs32 = nbrs.astype(jnp.int32)
    if n_pad != n:
        nbrs32 = jnp.concatenate(
            [nbrs32, jnp.zeros((n_pad - n, deg), jnp.int32)], axis=0)
    nbrs_flat = nbrs32.reshape(n_pad * deg)

    sc = _make_sc_trimmed_mean(n_pad, d_out, deg, remove)
    out = sc(h, nbrs_flat)
    return out[:n]


# speedup vs baseline: 1.0816x; 1.0816x over previous
import functools
import math

import jax
import jax.numpy as jnp
from jax import lax
from jax.experimental import pallas as pl
from jax.experimental.pallas import tpu as pltpu
from jax.experimental.pallas import tpu_sc as plsc

TPERC = 0.45
LANES = 16
NC, NS = 2, 16
NW = NC * NS
RING = 2
PER_W_BY_CORE = (240, 392)


def _selection_network(deg, out_lo, out_hi):
    net = []

    def merge(lo, n, r):
        m = r * 2
        if m < n:
            merge(lo, n, m)
            merge(lo + r, n, m)
            for i in range(lo + r, lo + n - r, m):
                net.append((i, i + r))
        else:
            net.append((lo, lo + r))

    def sort(lo, n):
        if n > 1:
            m = n // 2
            sort(lo, m)
            sort(lo + m, m)
            merge(lo, n, 1)

    sort(0, deg)

    state = {i: None for i in range(deg)}
    for w in range(out_lo, out_hi):
        state[w] = 'sum'
    kept = []
    for (i, j) in reversed(net):
        si, sj = state[i], state[j]
        if si is None and sj is None:
            continue
        if si == 'sum' and sj == 'sum':
            continue
        kept.append((i, j, si is not None, sj is not None))
        state[i] = 'exact'
        state[j] = 'exact'
    kept.reverse()
    return kept


def _matmul_body(x_ref, w_ref, h_ref):
    h_ref[...] = lax.dot_general(
        x_ref[...], w_ref[...],
        dimension_numbers=(((1,), (1,)), ((), ())),
        preferred_element_type=jnp.float32,
    )


def _linear(x, W):
    n, d_in = x.shape
    d_out = W.shape[0]
    blk = 1000
    assert n % blk == 0
    return pl.pallas_call(
        _matmul_body,
        grid=(n // blk,),
        in_specs=[
            pl.BlockSpec((blk, d_in), lambda i: (i, 0)),
            pl.BlockSpec((d_out, d_in), lambda i: (0, 0)),
        ],
        out_specs=pl.BlockSpec((blk, d_out), lambda i: (i, 0)),
        out_shape=jax.ShapeDtypeStruct((n, d_out), jnp.float32),
    )(x, W)


def _make_sc_trimmed_mean(n_pad, d, deg, remove):
    keep = deg - 2 * remove
    net = _selection_network(deg, remove, deg - remove)
    nb = max(1, 128 // deg)
    p0, p1 = PER_W_BY_CORE
    assert NS * (p0 + p1) == n_pad
    per_w_max = max(p0, p1)
    for p_c in (p0, p1):
        assert p_c % (nb * RING) == 0
    chunks = d // LANES
    idx_per_batch = nb * deg

    mesh = plsc.VectorSubcoreMesh(core_axis_name="c", subcore_axis_name="s")

    scratch = [pltpu.VMEM((per_w_max * deg,), jnp.int32)]
    scratch += [pltpu.VMEM((idx_per_batch,), jnp.int32) for _ in range(RING)]
    scratch += [pltpu.VMEM((idx_per_batch, d), jnp.float32)
                for _ in range(RING)]
    scratch += [pltpu.VMEM((per_w_max, d), jnp.float32)]
    scratch += [pltpu.SemaphoreType.DMA for _ in range(RING)]

    @functools.partial(
        pl.kernel,
        mesh=mesh,
        out_type=jax.ShapeDtypeStruct((n_pad, d), jnp.float32),
        scratch_types=scratch,
    )
    def sc_kernel(h_hbm, nbrs_hbm, out_hbm, idx_all, *rest):
        ibs = rest[0:RING]
        rows_bufs = rest[RING:2 * RING]
        out_all = rest[2 * RING]
        sems = rest[2 * RING + 1:]

        cid = lax.axis_index("c")
        sid = lax.axis_index("s")

        def fill_idx(b, ib):
            for k in range(idx_per_batch // LANES):
                ib[pl.ds(k * LANES, LANES)] = (
                    idx_all[pl.ds(b * idx_per_batch + k * LANES, LANES)])

        def start_gather(ib, rows, sem):
            pltpu.make_async_copy(h_hbm.at[ib], rows, sem).start()

        def run_pipeline(node_base, per_w):
            nbatch = per_w // nb

            pltpu.sync_copy(
                nbrs_hbm.at[pl.ds(node_base * deg, per_w * deg)],
                idx_all.at[pl.ds(0, per_w * deg)])

            def compute_batch(b, rows):
                def chunk_body(t, carry):
                    i = t // chunks
                    c = (t % chunks) * LANES
                    v = [rows[i * deg + k, pl.ds(c, LANES)]
                         for k in range(deg)]
                    for (a, bb, need_min, need_max) in net:
                        lo, hi = v[a], v[bb]
                        if need_min:
                            v[a] = jnp.minimum(lo, hi)
                        if need_max:
                            v[bb] = jnp.maximum(lo, hi)
                    acc = v[remove]
                    for w in range(remove + 1, deg - remove):
                        acc = acc + v[w]
                    out_all[b * nb + i, pl.ds(c, LANES)] = acc * (1.0 / keep)
                    return carry
                lax.fori_loop(0, nb * chunks, chunk_body, 0)

            for pb in range(RING - 1):
                fill_idx(pb, ibs[pb])
                start_gather(ibs[pb], rows_bufs[pb], sems[pb])

            def outer(bi, carry):
                for p in range(RING):
                    b = bi * RING + p
                    pn = (p + RING - 1) % RING

                    @pl.when(b + RING - 1 < nbatch)
                    def _():
                        fill_idx(b + RING - 1, ibs[pn])
                        start_gather(ibs[pn], rows_bufs[pn], sems[pn])

                    pltpu.make_async_copy(
                        h_hbm.at[ibs[p]], rows_bufs[p], sems[p]).wait()
                    compute_batch(b, rows_bufs[p])
                return carry

            lax.fori_loop(0, nbatch // RING, outer, 0)

            pltpu.sync_copy(
                out_all.at[pl.ds(0, per_w)],
                out_hbm.at[pl.ds(node_base, per_w)])

        @pl.when(cid == 0)
        def _():
            run_pipeline(sid * p0, p0)

        @pl.when(cid == 1)
        def _():
            run_pipeline(NS * p0 + sid * p1, p1)

    return sc_kernel


@jax.jit
def kernel(x, nbrs, W):
    n, d_in = x.shape
    d_out = W.shape[0]
    deg = nbrs.shape[1]
    remove = math.floor(deg * TPERC)

    h = _linear(x, W)

    n_pad = NS * sum(PER_W_BY_CORE)
    assert n_pad >= n
    nbrs32 = nbrs.astype(jnp.int32)
    if n_pad != n:
        nbrs32 = jnp.concatenate(
            [nbrs32, jnp.zeros((n_pad - n, deg), jnp.int32)], axis=0)
    nbrs_flat = nbrs32.reshape(n_pad * deg)

    sc = _make_sc_trimmed_mean(n_pad, d_out, deg, remove)
    out = sc(h, nbrs_flat)
    return out[:n]


# speedup vs baseline: 1.1308x; 1.0455x over previous
import functools
import math

import jax
import jax.numpy as jnp
from jax import lax
from jax.experimental import pallas as pl
from jax.experimental.pallas import tpu as pltpu
from jax.experimental.pallas import tpu_sc as plsc

TPERC = 0.45
LANES = 16
NC, NS = 2, 16
NW = NC * NS
RING = 2
PER_W_BY_CORE = (272, 360)


def _selection_network(deg, out_lo, out_hi):
    net = []

    def merge(lo, n, r):
        m = r * 2
        if m < n:
            merge(lo, n, m)
            merge(lo + r, n, m)
            for i in range(lo + r, lo + n - r, m):
                net.append((i, i + r))
        else:
            net.append((lo, lo + r))

    def sort(lo, n):
        if n > 1:
            m = n // 2
            sort(lo, m)
            sort(lo + m, m)
            merge(lo, n, 1)

    sort(0, deg)

    state = {i: None for i in range(deg)}
    for w in range(out_lo, out_hi):
        state[w] = 'sum'
    kept = []
    for (i, j) in reversed(net):
        si, sj = state[i], state[j]
        if si is None and sj is None:
            continue
        if si == 'sum' and sj == 'sum':
            continue
        kept.append((i, j, si is not None, sj is not None))
        state[i] = 'exact'
        state[j] = 'exact'
    kept.reverse()
    return kept


def _matmul_body(x_ref, w_ref, h_ref):
    h_ref[...] = lax.dot_general(
        x_ref[...], w_ref[...],
        dimension_numbers=(((1,), (1,)), ((), ())),
        preferred_element_type=jnp.float32,
    )


def _linear(x, W):
    n, d_in = x.shape
    d_out = W.shape[0]
    blk = 1000
    assert n % blk == 0
    return pl.pallas_call(
        _matmul_body,
        grid=(n // blk,),
        in_specs=[
            pl.BlockSpec((blk, d_in), lambda i: (i, 0)),
            pl.BlockSpec((d_out, d_in), lambda i: (0, 0)),
        ],
        out_specs=pl.BlockSpec((blk, d_out), lambda i: (i, 0)),
        out_shape=jax.ShapeDtypeStruct((n, d_out), jnp.float32),
    )(x, W)


def _make_sc_trimmed_mean(n_pad, d, deg, remove):
    keep = deg - 2 * remove
    net = _selection_network(deg, remove, deg - remove)
    nb = max(1, 128 // deg)
    p0, p1 = PER_W_BY_CORE
    assert NS * (p0 + p1) == n_pad
    per_w_max = max(p0, p1)
    for p_c in (p0, p1):
        assert p_c % (nb * RING) == 0
    chunks = d // LANES
    idx_per_batch = nb * deg

    mesh = plsc.VectorSubcoreMesh(core_axis_name="c", subcore_axis_name="s")

    scratch = [pltpu.VMEM((per_w_max * deg,), jnp.int32)]
    scratch += [pltpu.VMEM((idx_per_batch,), jnp.int32) for _ in range(RING)]
    scratch += [pltpu.VMEM((idx_per_batch, d), jnp.float32)
                for _ in range(RING)]
    scratch += [pltpu.VMEM((per_w_max, d), jnp.float32)]
    scratch += [pltpu.SemaphoreType.DMA for _ in range(RING)]

    @functools.partial(
        pl.kernel,
        mesh=mesh,
        out_type=jax.ShapeDtypeStruct((n_pad, d), jnp.float32),
        scratch_types=scratch,
    )
    def sc_kernel(h_hbm, nbrs_hbm, out_hbm, idx_all, *rest):
        ibs = rest[0:RING]
        rows_bufs = rest[RING:2 * RING]
        out_all = rest[2 * RING]
        sems = rest[2 * RING + 1:]

        cid = lax.axis_index("c")
        sid = lax.axis_index("s")

        def fill_idx(b, ib):
            for k in range(idx_per_batch // LANES):
                ib[pl.ds(k * LANES, LANES)] = (
                    idx_all[pl.ds(b * idx_per_batch + k * LANES, LANES)])

        def start_gather(ib, rows, sem):
            pltpu.make_async_copy(h_hbm.at[ib], rows, sem).start()

        def run_pipeline(node_base, per_w):
            nbatch = per_w // nb

            pltpu.sync_copy(
                nbrs_hbm.at[pl.ds(node_base * deg, per_w * deg)],
                idx_all.at[pl.ds(0, per_w * deg)])

            def compute_batch(b, rows):
                def chunk_body(t, carry):
                    i = t // chunks
                    c = (t % chunks) * LANES
                    v = [rows[i * deg + k, pl.ds(c, LANES)]
                         for k in range(deg)]
                    for (a, bb, need_min, need_max) in net:
                        lo, hi = v[a], v[bb]
                        if need_min:
                            v[a] = jnp.minimum(lo, hi)
                        if need_max:
                            v[bb] = jnp.maximum(lo, hi)
                    acc = v[remove]
                    for w in range(remove + 1, deg - remove):
                        acc = acc + v[w]
                    out_all[b * nb + i, pl.ds(c, LANES)] = acc * (1.0 / keep)
                    return carry
                lax.fori_loop(0, nb * chunks, chunk_body, 0)

            for pb in range(RING - 1):
                fill_idx(pb, ibs[pb])
                start_gather(ibs[pb], rows_bufs[pb], sems[pb])

            def outer(bi, carry):
                for p in range(RING):
                    b = bi * RING + p
                    pn = (p + RING - 1) % RING

                    @pl.when(b + RING - 1 < nbatch)
                    def _():
                        fill_idx(b + RING - 1, ibs[pn])
                        start_gather(ibs[pn], rows_bufs[pn], sems[pn])

                    pltpu.make_async_copy(
                        h_hbm.at[ibs[p]], rows_bufs[p], sems[p]).wait()
                    compute_batch(b, rows_bufs[p])
                return carry

            lax.fori_loop(0, nbatch // RING, outer, 0)

            pltpu.sync_copy(
                out_all.at[pl.ds(0, per_w)],
                out_hbm.at[pl.ds(node_base, per_w)])

        @pl.when(cid == 0)
        def _():
            run_pipeline(sid * p0, p0)

        @pl.when(cid == 1)
        def _():
            run_pipeline(NS * p0 + sid * p1, p1)

    return sc_kernel


@jax.jit
def kernel(x, nbrs, W):
    n, d_in = x.shape
    d_out = W.shape[0]
    deg = nbrs.shape[1]
    remove = math.floor(deg * TPERC)

    h = _linear(x, W)

    n_pad = NS * sum(PER_W_BY_CORE)
    assert n_pad >= n
    nbrs32 = nbrs.astype(jnp.int32)
    if n_pad != n:
        nbrs32 = jnp.concatenate(
            [nbrs32, jnp.zeros((n_pad - n, deg), jnp.int32)], axis=0)
    nbrs_flat = nbrs32.reshape(n_pad * deg)

    sc = _make_sc_trimmed_mean(n_pad, d_out, deg, remove)
    out = sc(h, nbrs_flat)
    return out[:n]


# speedup vs baseline: 1.1658x; 1.0310x over previous
import functools
import math

import jax
import jax.numpy as jnp
from jax import lax
from jax.experimental import pallas as pl
from jax.experimental.pallas import tpu as pltpu
from jax.experimental.pallas import tpu_sc as plsc

TPERC = 0.45
LANES = 16
NC, NS = 2, 16
NW = NC * NS
RING = 2
PER_W_BY_CORE = (296, 336)


def _selection_network(deg, out_lo, out_hi):
    net = []

    def merge(lo, n, r):
        m = r * 2
        if m < n:
            merge(lo, n, m)
            merge(lo + r, n, m)
            for i in range(lo + r, lo + n - r, m):
                net.append((i, i + r))
        else:
            net.append((lo, lo + r))

    def sort(lo, n):
        if n > 1:
            m = n // 2
            sort(lo, m)
            sort(lo + m, m)
            merge(lo, n, 1)

    sort(0, deg)

    state = {i: None for i in range(deg)}
    for w in range(out_lo, out_hi):
        state[w] = 'sum'
    kept = []
    for (i, j) in reversed(net):
        si, sj = state[i], state[j]
        if si is None and sj is None:
            continue
        if si == 'sum' and sj == 'sum':
            continue
        kept.append((i, j, si is not None, sj is not None))
        state[i] = 'exact'
        state[j] = 'exact'
    kept.reverse()
    return kept


def _matmul_body(x_ref, w_ref, h_ref):
    h_ref[...] = lax.dot_general(
        x_ref[...], w_ref[...],
        dimension_numbers=(((1,), (1,)), ((), ())),
        preferred_element_type=jnp.float32,
    )


def _linear(x, W):
    n, d_in = x.shape
    d_out = W.shape[0]
    blk = 1000
    assert n % blk == 0
    return pl.pallas_call(
        _matmul_body,
        grid=(n // blk,),
        in_specs=[
            pl.BlockSpec((blk, d_in), lambda i: (i, 0)),
            pl.BlockSpec((d_out, d_in), lambda i: (0, 0)),
        ],
        out_specs=pl.BlockSpec((blk, d_out), lambda i: (i, 0)),
        out_shape=jax.ShapeDtypeStruct((n, d_out), jnp.float32),
    )(x, W)


def _make_sc_trimmed_mean(n_pad, d, deg, remove):
    keep = deg - 2 * remove
    net = _selection_network(deg, remove, deg - remove)
    nb = max(1, 128 // deg)
    p0, p1 = PER_W_BY_CORE
    assert NS * (p0 + p1) == n_pad
    per_w_max = max(p0, p1)
    for p_c in (p0, p1):
        assert p_c % (nb * RING) == 0
    chunks = d // LANES
    idx_per_batch = nb * deg

    mesh = plsc.VectorSubcoreMesh(core_axis_name="c", subcore_axis_name="s")

    scratch = [pltpu.VMEM((per_w_max * deg,), jnp.int32)]
    scratch += [pltpu.VMEM((idx_per_batch,), jnp.int32) for _ in range(RING)]
    scratch += [pltpu.VMEM((idx_per_batch, d), jnp.float32)
                for _ in range(RING)]
    scratch += [pltpu.VMEM((per_w_max, d), jnp.float32)]
    scratch += [pltpu.SemaphoreType.DMA for _ in range(RING)]

    @functools.partial(
        pl.kernel,
        mesh=mesh,
        out_type=jax.ShapeDtypeStruct((n_pad, d), jnp.float32),
        scratch_types=scratch,
    )
    def sc_kernel(h_hbm, nbrs_hbm, out_hbm, idx_all, *rest):
        ibs = rest[0:RING]
        rows_bufs = rest[RING:2 * RING]
        out_all = rest[2 * RING]
        sems = rest[2 * RING + 1:]

        cid = lax.axis_index("c")
        sid = lax.axis_index("s")

        def fill_idx(b, ib):
            for k in range(idx_per_batch // LANES):
                ib[pl.ds(k * LANES, LANES)] = (
                    idx_all[pl.ds(b * idx_per_batch + k * LANES, LANES)])

        def start_gather(ib, rows, sem):
            pltpu.make_async_copy(h_hbm.at[ib], rows, sem).start()

        def run_pipeline(node_base, per_w):
            nbatch = per_w // nb

            pltpu.sync_copy(
                nbrs_hbm.at[pl.ds(node_base * deg, per_w * deg)],
                idx_all.at[pl.ds(0, per_w * deg)])

            def compute_batch(b, rows):
                def chunk_body(t, carry):
                    i = t // chunks
                    c = (t % chunks) * LANES
                    v = [rows[i * deg + k, pl.ds(c, LANES)]
                         for k in range(deg)]
                    for (a, bb, need_min, need_max) in net:
                        lo, hi = v[a], v[bb]
                        if need_min:
                            v[a] = jnp.minimum(lo, hi)
                        if need_max:
                            v[bb] = jnp.maximum(lo, hi)
                    acc = v[remove]
                    for w in range(remove + 1, deg - remove):
                        acc = acc + v[w]
                    out_all[b * nb + i, pl.ds(c, LANES)] = acc * (1.0 / keep)
                    return carry
                lax.fori_loop(0, nb * chunks, chunk_body, 0)

            for pb in range(RING - 1):
                fill_idx(pb, ibs[pb])
                start_gather(ibs[pb], rows_bufs[pb], sems[pb])

            def outer(bi, carry):
                for p in range(RING):
                    b = bi * RING + p
                    pn = (p + RING - 1) % RING

                    @pl.when(b + RING - 1 < nbatch)
                    def _():
                        fill_idx(b + RING - 1, ibs[pn])
                        start_gather(ibs[pn], rows_bufs[pn], sems[pn])

                    pltpu.make_async_copy(
                        h_hbm.at[ibs[p]], rows_bufs[p], sems[p]).wait()
                    compute_batch(b, rows_bufs[p])
                return carry

            lax.fori_loop(0, nbatch // RING, outer, 0)

            pltpu.sync_copy(
                out_all.at[pl.ds(0, per_w)],
                out_hbm.at[pl.ds(node_base, per_w)])

        @pl.when(cid == 0)
        def _():
            run_pipeline(sid * p0, p0)

        @pl.when(cid == 1)
        def _():
            run_pipeline(NS * p0 + sid * p1, p1)

    return sc_kernel


@jax.jit
def kernel(x, nbrs, W):
    n, d_in = x.shape
    d_out = W.shape[0]
    deg = nbrs.shape[1]
    remove = math.floor(deg * TPERC)

    h = _linear(x, W)

    n_pad = NS * sum(PER_W_BY_CORE)
    assert n_pad >= n
    nbrs32 = nbrs.astype(jnp.int32)
    if n_pad != n:
        nbrs32 = jnp.concatenate(
            [nbrs32, jnp.zeros((n_pad - n, deg), jnp.int32)], axis=0)
    nbrs_flat = nbrs32.reshape(n_pad * deg)

    sc = _make_sc_trimmed_mean(n_pad, d_out, deg, remove)
    out = sc(h, nbrs_flat)
    return out[:n]


# speedup vs baseline: 1.1899x; 1.0207x over previous
import functools
import math

import jax
import jax.numpy as jnp
from jax import lax
from jax.experimental import pallas as pl
from jax.experimental.pallas import tpu as pltpu
from jax.experimental.pallas import tpu_sc as plsc

TPERC = 0.45
LANES = 16
NC, NS = 2, 16
NW = NC * NS
RING = 2
PER_W_BY_CORE = (312, 320)


def _selection_network(deg, out_lo, out_hi):
    net = []

    def merge(lo, n, r):
        m = r * 2
        if m < n:
            merge(lo, n, m)
            merge(lo + r, n, m)
            for i in range(lo + r, lo + n - r, m):
                net.append((i, i + r))
        else:
            net.append((lo, lo + r))

    def sort(lo, n):
        if n > 1:
            m = n // 2
            sort(lo, m)
            sort(lo + m, m)
            merge(lo, n, 1)

    sort(0, deg)

    state = {i: None for i in range(deg)}
    for w in range(out_lo, out_hi):
        state[w] = 'sum'
    kept = []
    for (i, j) in reversed(net):
        si, sj = state[i], state[j]
        if si is None and sj is None:
            continue
        if si == 'sum' and sj == 'sum':
            continue
        kept.append((i, j, si is not None, sj is not None))
        state[i] = 'exact'
        state[j] = 'exact'
    kept.reverse()
    return kept


def _matmul_body(x_ref, w_ref, h_ref):
    h_ref[...] = lax.dot_general(
        x_ref[...], w_ref[...],
        dimension_numbers=(((1,), (1,)), ((), ())),
        preferred_element_type=jnp.float32,
    )


def _linear(x, W):
    n, d_in = x.shape
    d_out = W.shape[0]
    blk = 1000
    assert n % blk == 0
    return pl.pallas_call(
        _matmul_body,
        grid=(n // blk,),
        in_specs=[
            pl.BlockSpec((blk, d_in), lambda i: (i, 0)),
            pl.BlockSpec((d_out, d_in), lambda i: (0, 0)),
        ],
        out_specs=pl.BlockSpec((blk, d_out), lambda i: (i, 0)),
        out_shape=jax.ShapeDtypeStruct((n, d_out), jnp.float32),
    )(x, W)


def _make_sc_trimmed_mean(n_pad, d, deg, remove):
    keep = deg - 2 * remove
    net = _selection_network(deg, remove, deg - remove)
    nb = max(1, 128 // deg)
    p0, p1 = PER_W_BY_CORE
    assert NS * (p0 + p1) == n_pad
    per_w_max = max(p0, p1)
    for p_c in (p0, p1):
        assert p_c % (nb * RING) == 0
    chunks = d // LANES
    idx_per_batch = nb * deg

    mesh = plsc.VectorSubcoreMesh(core_axis_name="c", subcore_axis_name="s")

    scratch = [pltpu.VMEM((per_w_max * deg,), jnp.int32)]
    scratch += [pltpu.VMEM((idx_per_batch,), jnp.int32) for _ in range(RING)]
    scratch += [pltpu.VMEM((idx_per_batch, d), jnp.float32)
                for _ in range(RING)]
    scratch += [pltpu.VMEM((per_w_max, d), jnp.float32)]
    scratch += [pltpu.SemaphoreType.DMA for _ in range(RING)]

    @functools.partial(
        pl.kernel,
        mesh=mesh,
        out_type=jax.ShapeDtypeStruct((n_pad, d), jnp.float32),
        scratch_types=scratch,
    )
    def sc_kernel(h_hbm, nbrs_hbm, out_hbm, idx_all, *rest):
        ibs = rest[0:RING]
        rows_bufs = rest[RING:2 * RING]
        out_all = rest[2 * RING]
        sems = rest[2 * RING + 1:]

        cid = lax.axis_index("c")
        sid = lax.axis_index("s")

        def fill_idx(b, ib):
            for k in range(idx_per_batch // LANES):
                ib[pl.ds(k * LANES, LANES)] = (
                    idx_all[pl.ds(b * idx_per_batch + k * LANES, LANES)])

        def start_gather(ib, rows, sem):
            pltpu.make_async_copy(h_hbm.at[ib], rows, sem).start()

        def run_pipeline(node_base, per_w):
            nbatch = per_w // nb

            pltpu.sync_copy(
                nbrs_hbm.at[pl.ds(node_base * deg, per_w * deg)],
                idx_all.at[pl.ds(0, per_w * deg)])

            def compute_batch(b, rows):
                def chunk_body(t, carry):
                    i = t // chunks
                    c = (t % chunks) * LANES
                    v = [rows[i * deg + k, pl.ds(c, LANES)]
                         for k in range(deg)]
                    for (a, bb, need_min, need_max) in net:
                        lo, hi = v[a], v[bb]
                        if need_min:
                            v[a] = jnp.minimum(lo, hi)
                        if need_max:
                            v[bb] = jnp.maximum(lo, hi)
                    acc = v[remove]
                    for w in range(remove + 1, deg - remove):
                        acc = acc + v[w]
                    out_all[b * nb + i, pl.ds(c, LANES)] = acc * (1.0 / keep)
                    return carry
                lax.fori_loop(0, nb * chunks, chunk_body, 0)

            for pb in range(RING - 1):
                fill_idx(pb, ibs[pb])
                start_gather(ibs[pb], rows_bufs[pb], sems[pb])

            def outer(bi, carry):
                for p in range(RING):
                    b = bi * RING + p
                    pn = (p + RING - 1) % RING

                    @pl.when(b + RING - 1 < nbatch)
                    def _():
                        fill_idx(b + RING - 1, ibs[pn])
                        start_gather(ibs[pn], rows_bufs[pn], sems[pn])

                    pltpu.make_async_copy(
                        h_hbm.at[ibs[p]], rows_bufs[p], sems[p]).wait()
                    compute_batch(b, rows_bufs[p])
                return carry

            lax.fori_loop(0, nbatch // RING, outer, 0)

            pltpu.sync_copy(
                out_all.at[pl.ds(0, per_w)],
                out_hbm.at[pl.ds(node_base, per_w)])

        @pl.when(cid == 0)
        def _():
            run_pipeline(sid * p0, p0)

        @pl.when(cid == 1)
        def _():
            run_pipeline(NS * p0 + sid * p1, p1)

    return sc_kernel


@jax.jit
def kernel(x, nbrs, W):
    n, d_in = x.shape
    d_out = W.shape[0]
    deg = nbrs.shape[1]
    remove = math.floor(deg * TPERC)

    h = _linear(x, W)

    n_pad = NS * sum(PER_W_BY_CORE)
    assert n_pad >= n
    nbrs32 = nbrs.astype(jnp.int32)
    if n_pad != n:
        nbrs32 = jnp.concatenate(
            [nbrs32, jnp.zeros((n_pad - n, deg), jnp.int32)], axis=0)
    nbrs_flat = nbrs32.reshape(n_pad * deg)

    sc = _make_sc_trimmed_mean(n_pad, d_out, deg, remove)
    out = sc(h, nbrs_flat)
    return out[:n]
